# baseline (speedup 1.0000x reference)
import functools

import jax
import jax.numpy as jnp
from jax import lax
from jax.experimental import pallas as pl
from jax.experimental.pallas import tpu as pltpu, tpu_sc as plsc

_NC = 2
_NS = 16
_NW = _NC * _NS
_K = 80
_BN = 1000


def _layout(n_nodes):
    rpt = -(-n_nodes // _NS)
    rpt = -(-rpt // 8) * 8
    npad = rpt * _NS
    chunks = []
    off = 0
    while off < rpt:
        w = min(_K, rpt - off)
        chunks.append((off, w))
        off += w
    return rpt, npad, tuple(chunks)


@functools.lru_cache(maxsize=None)
def _sc_agg(n_nodes, n_edges, d):
    epw = n_edges // _NW
    nchunk = epw // _K
    rpt, npad, chunks = _layout(n_nodes)
    assert epw * _NW == n_edges and nchunk * _K == epw
    assert nchunk % 2 == 1
    npair = (nchunk - 1) // 2

    mesh = plsc.VectorSubcoreMesh(core_axis_name="c", subcore_axis_name="s")
    out_type = jax.ShapeDtypeStruct((_NC * npad, d), jnp.float32)
    scratch = [
        pltpu.VMEM((_K, d), jnp.float32),
        pltpu.VMEM((_K, d), jnp.float32),
        pltpu.VMEM((_K,), jnp.int32),
        pltpu.VMEM((_K,), jnp.int32),
        pltpu.VMEM((_K,), jnp.int32),
        pltpu.VMEM((_K,), jnp.int32),
        pltpu.VMEM_SHARED((npad, d), jnp.float32),
        pltpu.SemaphoreType.DMA,
        pltpu.SemaphoreType.DMA,
    ]

    def body(t_h, src_h, dst_h, z_h, out_h,
             rowsA, rowsB, sA, dA, sB, dB, acc, semA, semB):
        c = lax.axis_index("c")
        s = lax.axis_index("s")
        wid = s * _NC + c
        r0 = s * rpt

        for (co, cw) in chunks:
            pltpu.sync_copy(z_h.at[pl.ds(r0 + co, cw)],
                            rowsA.at[pl.ds(0, cw)])
            pltpu.sync_copy(rowsA.at[pl.ds(0, cw)],
                            acc.at[pl.ds(r0 + co, cw)])
        plsc.subcore_barrier()

        base = wid * epw

        def idx_load(g, sv, dv):
            pltpu.sync_copy(src_h.at[pl.ds(base + g * _K, _K)], sv)
            pltpu.sync_copy(dst_h.at[pl.ds(base + g * _K, _K)], dv)

        idx_load(0, sA, dA)
        pltpu.async_copy(t_h.at[sA], rowsA, semA)

        def pair(i, carry):
            idx_load(2 * i + 1, sB, dB)
            gB = pltpu.async_copy(t_h.at[sB], rowsB, semB)
            pltpu.make_async_copy(t_h.at[sA], rowsA, semA).wait()
            pltpu.sync_copy(rowsA, acc.at[dA], add=True)
            idx_load(2 * i + 2, sA, dA)
            pltpu.async_copy(t_h.at[sA], rowsA, semA)
            gB.wait()
            pltpu.sync_copy(rowsB, acc.at[dB], add=True)
            return carry
        lax.fori_loop(0, npair, pair, 0)
        pltpu.make_async_copy(t_h.at[sA], rowsA, semA).wait()
        pltpu.sync_copy(rowsA, acc.at[dA], add=True)

        plsc.subcore_barrier()
        for (co, cw) in chunks:
            pltpu.sync_copy(acc.at[pl.ds(r0 + co, cw)],
                            rowsA.at[pl.ds(0, cw)])
            pltpu.sync_copy(rowsA.at[pl.ds(0, cw)],
                            out_h.at[pl.ds(c * npad + r0 + co, cw)])

    return pl.kernel(body, out_type=out_type, mesh=mesh,
                     scratch_types=scratch)


@functools.lru_cache(maxsize=None)
def _sc_count(n_nodes, n_edges):
    d = 128
    epw = n_edges // _NW
    nchunk = epw // _K
    rpt, npad, chunks = _layout(n_nodes)

    mesh = plsc.VectorSubcoreMesh(core_axis_name="c", subcore_axis_name="s")
    out_type = jax.ShapeDtypeStruct((_NC * npad, d), jnp.float32)
    scratch = [
        pltpu.VMEM((_K,), jnp.int32),
        pltpu.VMEM((_K, d), jnp.float32),
        pltpu.VMEM((_K, d), jnp.float32),
        pltpu.VMEM_SHARED((npad, d), jnp.float32),
    ]

    def body(dst_h, z_h, ones_h, out_h, dstv, rows, ones, acc):
        c = lax.axis_index("c")
        s = lax.axis_index("s")
        wid = s * _NC + c
        r0 = s * rpt

        for (co, cw) in chunks:
            pltpu.sync_copy(z_h.at[pl.ds(r0 + co, cw)],
                            rows.at[pl.ds(0, cw)])
            pltpu.sync_copy(rows.at[pl.ds(0, cw)],
                            acc.at[pl.ds(r0 + co, cw)])
        pltpu.sync_copy(ones_h, ones)
        plsc.subcore_barrier()

        base = wid * epw

        def chunk(g, carry):
            off = base + g * _K
            pltpu.sync_copy(dst_h.at[pl.ds(off, _K)], dstv)
            pltpu.sync_copy(ones, acc.at[dstv], add=True)
            return carry
        lax.fori_loop(0, nchunk, chunk, 0)

        plsc.subcore_barrier()
        for (co, cw) in chunks:
            pltpu.sync_copy(acc.at[pl.ds(r0 + co, cw)],
                            rows.at[pl.ds(0, cw)])
            pltpu.sync_copy(rows.at[pl.ds(0, cw)],
                            out_h.at[pl.ds(c * npad + r0 + co, cw)])

    return pl.kernel(body, out_type=out_type, mesh=mesh,
                     scratch_types=scratch)


def _tc_in_body(x_ref, wl_ref, bl_ref, wr_ref, t_ref, r_ref):
    xb = x_ref[...]
    t_ref[...] = jnp.dot(xb, wl_ref[...], preferred_element_type=jnp.float32,
                         precision=lax.Precision.HIGHEST)
    r_ref[...] = (jnp.dot(xb, wr_ref[...], preferred_element_type=jnp.float32,
                          precision=lax.Precision.HIGHEST)
                  + bl_ref[...])


def _tc_comb1_body(p0, p1, c0, c1, r, wl, bl, wr, t_o, r_o, inv_o):
    inv = 1.0 / jnp.maximum(c0[...] + c1[...], 1.0)
    h = jnp.maximum((p0[...] + p1[...]) * inv + r[...], 0.0)
    t_o[...] = jnp.dot(h, wl[...], preferred_element_type=jnp.float32,
                       precision=lax.Precision.HIGHEST)
    r_o[...] = (jnp.dot(h, wr[...], preferred_element_type=jnp.float32,
                        precision=lax.Precision.HIGHEST)
                + bl[...])
    inv_o[...] = inv


def _tc_comb2_body(p0, p1, inv_ref, r, wl, bl, wr, t_o, r_o):
    inv = inv_ref[...]
    h = jnp.maximum((p0[...] + p1[...]) * inv + r[...], 0.0)
    t_o[...] = jnp.dot(h, wl[...], preferred_element_type=jnp.float32,
                       precision=lax.Precision.HIGHEST)
    r_o[...] = (jnp.dot(h, wr[...], preferred_element_type=jnp.float32,
                        precision=lax.Precision.HIGHEST)
                + bl[...])


def _tc_final_body(p0, p1, inv_ref, r, o):
    o[...] = (p0[...] + p1[...]) * inv_ref[...] + r[...]


def _row_spec(w):
    return pl.BlockSpec((_BN, w), lambda i: (i, 0))


def _rep_spec(shape):
    return pl.BlockSpec(shape, lambda i: (0, 0))


def _tc_in(x, wl, bl, wr):
    n, d = x.shape
    return pl.pallas_call(
        _tc_in_body,
        grid=(n // _BN,),
        in_specs=[_row_spec(d), _rep_spec((d, d)), _rep_spec((1, d)),
                  _rep_spec((d, d))],
        out_specs=[_row_spec(d), _row_spec(d)],
        out_shape=[jax.ShapeDtypeStruct((n, d), jnp.float32)] * 2,
    )(x, wl, bl.reshape(1, d), wr)


def _tc_comb1(p0, p1, c0, c1, r, wl, bl, wr):
    n, d = r.shape
    do = wl.shape[1]
    return pl.pallas_call(
        _tc_comb1_body,
        grid=(n // _BN,),
        in_specs=[_row_spec(d), _row_spec(d), _row_spec(1), _row_spec(1),
                  _row_spec(d), _rep_spec((d, do)), _rep_spec((1, do)),
                  _rep_spec((d, do))],
        out_specs=[_row_spec(do), _row_spec(do), _row_spec(1)],
        out_shape=[jax.ShapeDtypeStruct((n, do), jnp.float32)] * 2
        + [jax.ShapeDtypeStruct((n, 1), jnp.float32)],
    )(p0, p1, c0, c1, r, wl, bl.reshape(1, do), wr)


def _tc_comb2(p0, p1, inv, r, wl, bl, wr):
    n, d = r.shape
    do = wl.shape[1]
    return pl.pallas_call(
        _tc_comb2_body,
        grid=(n // _BN,),
        in_specs=[_row_spec(d), _row_spec(d), _row_spec(1), _row_spec(d),
                  _rep_spec((d, do)), _rep_spec((1, do)), _rep_spec((d, do))],
        out_specs=[_row_spec(do), _row_spec(do)],
        out_shape=[jax.ShapeDtypeStruct((n, do), jnp.float32)] * 2,
    )(p0, p1, inv, r, wl, bl.reshape(1, do), wr)


def _tc_final(p0, p1, inv, r):
    n, d = r.shape
    return pl.pallas_call(
        _tc_final_body,
        grid=(n // _BN,),
        in_specs=[_row_spec(d), _row_spec(d), _row_spec(1), _row_spec(d)],
        out_specs=_row_spec(d),
        out_shape=jax.ShapeDtypeStruct((n, d), jnp.float32),
    )(p0, p1, inv, r)


def kernel(x, edge_index, Wl1, bl1, Wr1, Wl2, bl2, Wr2, Wl3, bl3, Wr3):
    n, d = x.shape
    e = edge_index.shape[1]
    src = edge_index[0]
    dst = edge_index[1]
    _, npad, _ = _layout(n)
    zs = jnp.zeros((npad, d), jnp.float32)
    ones_k = jnp.ones((_K, 128), jnp.float32)

    cnt = _sc_count(n, e)(dst, zs, ones_k)
    t1, r1 = _tc_in(x, Wl1, bl1, Wr1)
    agg1 = _sc_agg(n, e, d)(t1, src, dst, zs)
    t2, r2, invc = _tc_comb1(agg1[:n], agg1[npad:npad + n],
                             cnt[:n, :1], cnt[npad:npad + n, :1],
                             r1, Wl2, bl2, Wr2)
    agg2 = _sc_agg(n, e, d)(t2, src, dst, zs)
    wl3p = jnp.pad(Wl3, ((0, 0), (0, d - Wl3.shape[1])))
    wr3p = jnp.pad(Wr3, ((0, 0), (0, d - Wr3.shape[1])))
    bl3p = jnp.pad(bl3, (0, d - bl3.shape[0]))
    t3, r3 = _tc_comb2(agg2[:n], agg2[npad:npad + n], invc, r2,
                       wl3p, bl3p, wr3p)
    agg3 = _sc_agg(n, e, d)(t3, src, dst, zs)
    out = _tc_final(agg3[:n], agg3[npad:npad + n], invc, r3)
    return out[:, :1]


# speedup vs baseline: 1.4796x; 1.4796x over previous
import functools

import jax
import jax.numpy as jnp
from jax import lax
from jax.experimental import pallas as pl
from jax.experimental.pallas import tpu as pltpu, tpu_sc as plsc

_NC = 2
_NS = 16
_NW = _NC * _NS
_K = 80
_BN = 1000


def _layout(n_nodes):
    rpt = -(-n_nodes // _NS)
    rpt = -(-rpt // 8) * 8
    npad = rpt * _NS
    chunks = []
    off = 0
    while off < rpt:
        w = min(_K, rpt - off)
        chunks.append((off, w))
        off += w
    return rpt, npad, tuple(chunks)


@functools.lru_cache(maxsize=None)
def _sc_agg(n_nodes, n_edges, d):
    epw = n_edges // _NW
    nchunk = epw // _K
    rpt, npad, chunks = _layout(n_nodes)
    assert epw * _NW == n_edges and nchunk * _K == epw
    assert nchunk % 2 == 1
    npair = (nchunk - 1) // 2

    mesh = plsc.VectorSubcoreMesh(core_axis_name="c", subcore_axis_name="s")
    out_type = jax.ShapeDtypeStruct((_NC * npad, d), jnp.float32)
    scratch = [
        pltpu.VMEM((_K, d), jnp.float32),
        pltpu.VMEM((_K, d), jnp.float32),
        pltpu.VMEM((epw,), jnp.int32),
        pltpu.VMEM((epw,), jnp.int32),
        pltpu.VMEM_SHARED((npad, d), jnp.float32),
        pltpu.SemaphoreType.DMA,
        pltpu.SemaphoreType.DMA,
        pltpu.SemaphoreType.DMA,
    ]

    def body(t_h, src_h, dst_h, z_h, out_h,
             rowsA, rowsB, s_all, d_all, acc, semA, semB, isem):
        c = lax.axis_index("c")
        s = lax.axis_index("s")
        wid = s * _NC + c
        r0 = s * rpt
        base = wid * epw

        pltpu.async_copy(src_h.at[pl.ds(base, epw)], s_all, isem)
        pltpu.async_copy(dst_h.at[pl.ds(base, epw)], d_all, isem)

        pltpu.sync_copy(z_h.at[pl.ds(0, _K)], rowsA)
        for (co, cw) in chunks:
            pltpu.sync_copy(rowsA.at[pl.ds(0, cw)],
                            acc.at[pl.ds(r0 + co, cw)])
        pltpu.make_async_copy(src_h.at[pl.ds(base, epw)], s_all, isem).wait()
        pltpu.make_async_copy(dst_h.at[pl.ds(base, epw)], d_all, isem).wait()
        plsc.subcore_barrier()

        def sidx(g):
            return s_all.at[pl.ds(g * _K, _K)]

        def didx(g):
            return d_all.at[pl.ds(g * _K, _K)]

        pltpu.async_copy(t_h.at[sidx(0)], rowsA, semA)

        def pair(i, carry):
            gB = pltpu.async_copy(t_h.at[sidx(2 * i + 1)], rowsB, semB)
            pltpu.make_async_copy(t_h.at[sidx(0)], rowsA, semA).wait()
            pltpu.sync_copy(rowsA, acc.at[didx(2 * i)], add=True)
            pltpu.async_copy(t_h.at[sidx(2 * i + 2)], rowsA, semA)
            gB.wait()
            pltpu.sync_copy(rowsB, acc.at[didx(2 * i + 1)], add=True)
            return carry
        lax.fori_loop(0, npair, pair, 0)
        pltpu.make_async_copy(t_h.at[sidx(0)], rowsA, semA).wait()
        pltpu.sync_copy(rowsA, acc.at[didx(nchunk - 1)], add=True)

        plsc.subcore_barrier()
        for (co, cw) in chunks:
            pltpu.sync_copy(acc.at[pl.ds(r0 + co, cw)],
                            rowsA.at[pl.ds(0, cw)])
            pltpu.sync_copy(rowsA.at[pl.ds(0, cw)],
                            out_h.at[pl.ds(c * npad + r0 + co, cw)])

    return pl.kernel(body, out_type=out_type, mesh=mesh,
                     scratch_types=scratch)


@functools.lru_cache(maxsize=None)
def _sc_count(n_nodes, n_edges):
    d = 128
    epw = n_edges // _NW
    nchunk = epw // _K
    rpt, npad, chunks = _layout(n_nodes)

    mesh = plsc.VectorSubcoreMesh(core_axis_name="c", subcore_axis_name="s")
    out_type = jax.ShapeDtypeStruct((_NC * npad, d), jnp.float32)
    scratch = [
        pltpu.VMEM((epw,), jnp.int32),
        pltpu.VMEM((_K, d), jnp.float32),
        pltpu.VMEM((_K, d), jnp.float32),
        pltpu.VMEM_SHARED((npad, d), jnp.float32),
        pltpu.SemaphoreType.DMA,
    ]

    def body(dst_h, z_h, ones_h, out_h, d_all, rows, ones, acc, isem):
        c = lax.axis_index("c")
        s = lax.axis_index("s")
        wid = s * _NC + c
        r0 = s * rpt
        base = wid * epw

        pltpu.async_copy(dst_h.at[pl.ds(base, epw)], d_all, isem)
        pltpu.sync_copy(z_h.at[pl.ds(0, _K)], rows)
        for (co, cw) in chunks:
            pltpu.sync_copy(rows.at[pl.ds(0, cw)],
                            acc.at[pl.ds(r0 + co, cw)])
        pltpu.sync_copy(ones_h, ones)
        pltpu.make_async_copy(dst_h.at[pl.ds(base, epw)], d_all, isem).wait()
        plsc.subcore_barrier()

        def chunk(g, carry):
            pltpu.sync_copy(ones, acc.at[d_all.at[pl.ds(g * _K, _K)]],
                            add=True)
            return carry
        lax.fori_loop(0, nchunk, chunk, 0)

        plsc.subcore_barrier()
        for (co, cw) in chunks:
            pltpu.sync_copy(acc.at[pl.ds(r0 + co, cw)],
                            rows.at[pl.ds(0, cw)])
            pltpu.sync_copy(rows.at[pl.ds(0, cw)],
                            out_h.at[pl.ds(c * npad + r0 + co, cw)])

    return pl.kernel(body, out_type=out_type, mesh=mesh,
                     scratch_types=scratch)


def _tc_in_body(x_ref, wl_ref, bl_ref, wr_ref, t_ref, r_ref):
    xb = x_ref[...]
    t_ref[...] = jnp.dot(xb, wl_ref[...], preferred_element_type=jnp.float32,
                         precision=lax.Precision.HIGHEST)
    r_ref[...] = (jnp.dot(xb, wr_ref[...], preferred_element_type=jnp.float32,
                          precision=lax.Precision.HIGHEST)
                  + bl_ref[...])


def _tc_comb1_body(p0, p1, c0, c1, r, wl, bl, wr, t_o, r_o, inv_o):
    inv = 1.0 / jnp.maximum(c0[...] + c1[...], 1.0)
    h = jnp.maximum((p0[...] + p1[...]) * inv + r[...], 0.0)
    t_o[...] = jnp.dot(h, wl[...], preferred_element_type=jnp.float32,
                       precision=lax.Precision.HIGHEST)
    r_o[...] = (jnp.dot(h, wr[...], preferred_element_type=jnp.float32,
                        precision=lax.Precision.HIGHEST)
                + bl[...])
    inv_o[...] = inv


def _tc_comb2_body(p0, p1, inv_ref, r, wl, bl, wr, t_o, r_o):
    inv = inv_ref[...]
    h = jnp.maximum((p0[...] + p1[...]) * inv + r[...], 0.0)
    t_o[...] = jnp.dot(h, wl[...], preferred_element_type=jnp.float32,
                       precision=lax.Precision.HIGHEST)
    r_o[...] = (jnp.dot(h, wr[...], preferred_element_type=jnp.float32,
                        precision=lax.Precision.HIGHEST)
                + bl[...])


def _tc_final_body(p0, p1, inv_ref, r, o):
    o[...] = (p0[...] + p1[...]) * inv_ref[...] + r[...]


def _row_spec(w):
    return pl.BlockSpec((_BN, w), lambda i: (i, 0))


def _rep_spec(shape):
    return pl.BlockSpec(shape, lambda i: (0, 0))


def _tc_in(x, wl, bl, wr):
    n, d = x.shape
    return pl.pallas_call(
        _tc_in_body,
        grid=(n // _BN,),
        in_specs=[_row_spec(d), _rep_spec((d, d)), _rep_spec((1, d)),
                  _rep_spec((d, d))],
        out_specs=[_row_spec(d), _row_spec(d)],
        out_shape=[jax.ShapeDtypeStruct((n, d), jnp.float32)] * 2,
    )(x, wl, bl.reshape(1, d), wr)


def _tc_comb1(p0, p1, c0, c1, r, wl, bl, wr):
    n, d = r.shape
    do = wl.shape[1]
    return pl.pallas_call(
        _tc_comb1_body,
        grid=(n // _BN,),
        in_specs=[_row_spec(d), _row_spec(d), _row_spec(1), _row_spec(1),
                  _row_spec(d), _rep_spec((d, do)), _rep_spec((1, do)),
                  _rep_spec((d, do))],
        out_specs=[_row_spec(do), _row_spec(do), _row_spec(1)],
        out_shape=[jax.ShapeDtypeStruct((n, do), jnp.float32)] * 2
        + [jax.ShapeDtypeStruct((n, 1), jnp.float32)],
    )(p0, p1, c0, c1, r, wl, bl.reshape(1, do), wr)


def _tc_comb2(p0, p1, inv, r, wl, bl, wr):
    n, d = r.shape
    do = wl.shape[1]
    return pl.pallas_call(
        _tc_comb2_body,
        grid=(n // _BN,),
        in_specs=[_row_spec(d), _row_spec(d), _row_spec(1), _row_spec(d),
                  _rep_spec((d, do)), _rep_spec((1, do)), _rep_spec((d, do))],
        out_specs=[_row_spec(do), _row_spec(do)],
        out_shape=[jax.ShapeDtypeStruct((n, do), jnp.float32)] * 2,
    )(p0, p1, inv, r, wl, bl.reshape(1, do), wr)


def _tc_final(p0, p1, inv, r):
    n, d = r.shape
    return pl.pallas_call(
        _tc_final_body,
        grid=(n // _BN,),
        in_specs=[_row_spec(d), _row_spec(d), _row_spec(1), _row_spec(d)],
        out_specs=_row_spec(d),
        out_shape=jax.ShapeDtypeStruct((n, d), jnp.float32),
    )(p0, p1, inv, r)


def kernel(x, edge_index, Wl1, bl1, Wr1, Wl2, bl2, Wr2, Wl3, bl3, Wr3):
    n, d = x.shape
    e = edge_index.shape[1]
    src = edge_index[0]
    dst = edge_index[1]
    _, npad, _ = _layout(n)
    zs = jnp.zeros((npad, d), jnp.float32)
    ones_k = jnp.ones((_K, 128), jnp.float32)

    cnt = _sc_count(n, e)(dst, zs, ones_k)
    t1, r1 = _tc_in(x, Wl1, bl1, Wr1)
    agg1 = _sc_agg(n, e, d)(t1, src, dst, zs)
    t2, r2, invc = _tc_comb1(agg1[:n], agg1[npad:npad + n],
                             cnt[:n, :1], cnt[npad:npad + n, :1],
                             r1, Wl2, bl2, Wr2)
    agg2 = _sc_agg(n, e, d)(t2, src, dst, zs)
    wl3p = jnp.pad(Wl3, ((0, 0), (0, d - Wl3.shape[1])))
    wr3p = jnp.pad(Wr3, ((0, 0), (0, d - Wr3.shape[1])))
    bl3p = jnp.pad(bl3, (0, d - bl3.shape[0]))
    t3, r3 = _tc_comb2(agg2[:n], agg2[npad:npad + n], invc, r2,
                       wl3p, bl3p, wr3p)
    agg3 = _sc_agg(n, e, d)(t3, src, dst, zs)
    out = _tc_final(agg3[:n], agg3[npad:npad + n], invc, r3)
    return out[:, :1]
